# baseline (speedup 1.0000x reference)
import functools

import jax
import jax.numpy as jnp
from jax import lax
from jax.experimental import pallas as pl
from jax.experimental.pallas import tpu as pltpu
from jax.experimental.pallas import tpu_sc as plsc

NC = 2
NS = 16
NW = NC * NS
K = 128


def _sc_segment_sum(h, src_r, dst_r, zeros_init, n_chunks):
  n, d = h.shape
  rows_per_sub = n // NS
  mesh = plsc.VectorSubcoreMesh(core_axis_name="c", subcore_axis_name="s")

  n_q = n_chunks // 4

  @functools.partial(
      pl.kernel,
      out_type=jax.ShapeDtypeStruct((NC, n, d), jnp.float32),
      mesh=mesh,
      scratch_types=[
          pltpu.VMEM((n_q, K), jnp.int32),
          pltpu.VMEM((n_q, K), jnp.int32),
          pltpu.VMEM((n_q, K), jnp.int32),
          pltpu.VMEM((n_q, K), jnp.int32),
          pltpu.VMEM((K, d), jnp.float32),
          pltpu.VMEM((K, d), jnp.float32),
          pltpu.VMEM_SHARED((n, d), jnp.float32),
          pltpu.SemaphoreType.DMA,
          pltpu.SemaphoreType.DMA,
          pltpu.SemaphoreType.DMA,
          pltpu.SemaphoreType.DMA,
          pltpu.SemaphoreType.DMA,
          pltpu.SemaphoreType.DMA,
      ],
  )
  def seg(h_hbm, src_hbm, dst_hbm, z_hbm, out_hbm, src_a, dst_a, src_b,
          dst_b, rows0, rows1, agg_sh, sia, sib, s0, s1, ss0, ss1):
    cid = lax.axis_index("c")
    sid = lax.axis_index("s")
    wid = sid * NC + cid
    pltpu.async_copy(src_hbm.at[wid, 0], src_a, sia)
    pltpu.async_copy(dst_hbm.at[wid, 0], dst_a, sia)
    pltpu.async_copy(src_hbm.at[wid, 1], src_b, sib)
    pltpu.async_copy(dst_hbm.at[wid, 1], dst_b, sib)
    pltpu.sync_copy(z_hbm, agg_sh.at[pl.ds(sid * rows_per_sub, rows_per_sub)])
    plsc.subcore_barrier()

    def process_quarter(q, src_q, dst_q, si_q):
      pltpu.make_async_copy(src_hbm.at[wid, q], src_q, si_q).wait()
      pltpu.make_async_copy(dst_hbm.at[wid, q], dst_q, si_q).wait()
      pltpu.async_copy(h_hbm.at[src_q.at[0]], rows0, s0)
      pltpu.make_async_copy(h_hbm.at[src_q.at[0]], rows0, s0).wait()
      pltpu.async_copy(h_hbm.at[src_q.at[1]], rows1, s1)
      pltpu.async_copy(rows0, agg_sh.at[dst_q.at[0]], ss0, add=True)

      @pl.loop(2, n_q, step=2)
      def _(j):
        pltpu.make_async_copy(h_hbm.at[src_q.at[j - 1]], rows1, s1).wait()
        pltpu.make_async_copy(rows0, agg_sh.at[dst_q.at[j - 2]], ss0).wait()
        pltpu.async_copy(h_hbm.at[src_q.at[j]], rows0, s0)
        pltpu.async_copy(rows1, agg_sh.at[dst_q.at[j - 1]], ss1, add=True)
        pltpu.make_async_copy(h_hbm.at[src_q.at[j]], rows0, s0).wait()
        pltpu.make_async_copy(rows1, agg_sh.at[dst_q.at[j - 1]], ss1).wait()
        pltpu.async_copy(h_hbm.at[src_q.at[j + 1]], rows1, s1)
        pltpu.async_copy(rows0, agg_sh.at[dst_q.at[j]], ss0, add=True)

      jl = n_q - 1
      pltpu.make_async_copy(h_hbm.at[src_q.at[jl]], rows1, s1).wait()
      pltpu.make_async_copy(rows0, agg_sh.at[dst_q.at[jl - 1]], ss0).wait()
      pltpu.async_copy(rows1, agg_sh.at[dst_q.at[jl]], ss1, add=True)
      pltpu.make_async_copy(rows1, agg_sh.at[dst_q.at[jl]], ss1).wait()

    process_quarter(0, src_a, dst_a, sia)
    pltpu.async_copy(src_hbm.at[wid, 2], src_a, sia)
    pltpu.async_copy(dst_hbm.at[wid, 2], dst_a, sia)
    process_quarter(1, src_b, dst_b, sib)
    pltpu.async_copy(src_hbm.at[wid, 3], src_b, sib)
    pltpu.async_copy(dst_hbm.at[wid, 3], dst_b, sib)
    process_quarter(2, src_a, dst_a, sia)
    process_quarter(3, src_b, dst_b, sib)

    plsc.subcore_barrier()
    sl = pl.ds(sid * rows_per_sub, rows_per_sub)
    pltpu.sync_copy(agg_sh.at[sl], out_hbm.at[cid, sl])

  return seg(h, src_r, dst_r, zeros_init)


def _tc_layer(h, partial, w1, b1, w2, b2, bn):
  n, d = h.shape
  grid = n // bn

  def body(h_ref, p_ref, w1_ref, b1_ref, w2_ref, b2_ref, o_ref):
    z = h_ref[...] + p_ref[0] + p_ref[1]
    z = jnp.maximum(
        jnp.dot(z, w1_ref[...], preferred_element_type=jnp.float32)
        + b1_ref[...], 0.0)
    o = jnp.dot(z, w2_ref[...], preferred_element_type=jnp.float32) \
        + b2_ref[...]
    o_ref[...] = jnp.maximum(o, 0.0)

  return pl.pallas_call(
      body,
      grid=(grid,),
      in_specs=[
          pl.BlockSpec((bn, d), lambda i: (i, 0)),
          pl.BlockSpec((NC, bn, d), lambda i: (0, i, 0)),
          pl.BlockSpec((d, d), lambda i: (0, 0)),
          pl.BlockSpec((1, d), lambda i: (0, 0)),
          pl.BlockSpec((d, d), lambda i: (0, 0)),
          pl.BlockSpec((1, d), lambda i: (0, 0)),
      ],
      out_specs=pl.BlockSpec((bn, d), lambda i: (i, 0)),
      out_shape=jax.ShapeDtypeStruct((n, d), jnp.float32),
  )(h, partial, w1, b1, w2, b2)


def _tc_layer4_pool(h, partial, w1, b1, w2, b2, batch_r, fc_w, fc_b, bn,
                    n_graphs):
  n, d = h.shape
  c = fc_w.shape[1]
  grid = n // bn

  def body(h_ref, p_ref, w1_ref, b1_ref, w2_ref, b2_ref, batch_ref,
           fcw_ref, fcb_ref, o_ref, sums_scr, cnts_scr):
    i = pl.program_id(0)

    @pl.when(i == 0)
    def _():
      sums_scr[...] = jnp.zeros_like(sums_scr)
      cnts_scr[...] = jnp.zeros_like(cnts_scr)

    z = h_ref[...] + p_ref[0] + p_ref[1]
    z = jnp.maximum(
        jnp.dot(z, w1_ref[...], preferred_element_type=jnp.float32)
        + b1_ref[...], 0.0)
    h4 = jnp.maximum(
        jnp.dot(z, w2_ref[...], preferred_element_type=jnp.float32)
        + b2_ref[...], 0.0)

    b2d = batch_ref[0]
    mask_t = (lax.broadcasted_iota(jnp.int32, (n_graphs, bn), 0)
              == b2d).astype(jnp.float32)
    sums_scr[...] += jnp.dot(mask_t, h4, preferred_element_type=jnp.float32)
    cnts_scr[...] += jnp.dot(mask_t, jnp.ones((bn, d), jnp.float32),
                             preferred_element_type=jnp.float32)

    @pl.when(i == grid - 1)
    def _():
      pooled = sums_scr[...] / jnp.maximum(cnts_scr[...], 1.0)
      logits = jnp.dot(pooled, fcw_ref[...],
                       preferred_element_type=jnp.float32) + fcb_ref[...]
      m = jnp.max(logits, axis=1, keepdims=True)
      e = jnp.exp(logits - m)
      s = jnp.sum(e, axis=1, keepdims=True)
      o_ref[...] = logits - m - jnp.log(s)

  return pl.pallas_call(
      body,
      grid=(grid,),
      in_specs=[
          pl.BlockSpec((bn, d), lambda i: (i, 0)),
          pl.BlockSpec((NC, bn, d), lambda i: (0, i, 0)),
          pl.BlockSpec((d, d), lambda i: (0, 0)),
          pl.BlockSpec((1, d), lambda i: (0, 0)),
          pl.BlockSpec((d, d), lambda i: (0, 0)),
          pl.BlockSpec((1, d), lambda i: (0, 0)),
          pl.BlockSpec((1, 1, bn), lambda i: (i, 0, 0)),
          pl.BlockSpec((d, c), lambda i: (0, 0)),
          pl.BlockSpec((1, c), lambda i: (0, 0)),
      ],
      out_specs=pl.BlockSpec((n_graphs, c), lambda i: (0, 0)),
      out_shape=jax.ShapeDtypeStruct((n_graphs, c), jnp.float32),
      scratch_shapes=[
          pltpu.VMEM((n_graphs, d), jnp.float32),
          pltpu.VMEM((n_graphs, d), jnp.float32),
      ],
  )(h, partial, w1, b1, w2, b2, batch_r, fc_w, fc_b)


def kernel(x, edge_index, batch,
           conv1_W1, conv1_b1, conv1_W2, conv1_b2,
           conv2_W1, conv2_b1, conv2_W2, conv2_b2,
           conv3_W1, conv3_b1, conv3_W2, conv3_b2,
           conv4_W1, conv4_b1, conv4_W2, conv4_b2,
           fc_W, fc_b):
  n, d = x.shape
  e = edge_index.shape[1]
  n_graphs = 64
  n_pad = ((n + NS * 8 * 4 - 1) // (NS * 8 * 4)) * (NS * 8 * 4)
  if n_pad == n:
    n_pad = n
  x = jnp.pad(x, ((0, n_pad - n), (0, 0)))
  batch = jnp.pad(batch.astype(jnp.int32), (0, n_pad - n),
                  constant_values=n_graphs)
  bn = n_pad // 10
  rows_per_sub = n_pad // NS

  n_chunks = -(-e // (NW * K))
  n_chunks = ((n_chunks + 7) // 8) * 8
  e_pad = NW * n_chunks * K
  src = edge_index[0].astype(jnp.int32)
  dst = edge_index[1].astype(jnp.int32)
  pad = e_pad - e
  if pad:
    r = jnp.arange(pad, dtype=jnp.int32)
    src = jnp.concatenate([src, r % n])
    dst = jnp.concatenate([dst, n + r % (n_pad - n)])
  src_r = src.reshape(NW, 4, n_chunks // 4, K)
  dst_r = dst.reshape(NW, 4, n_chunks // 4, K)

  zeros_init = jnp.zeros((rows_per_sub, d), jnp.float32)
  batch_r = batch.reshape(n_pad // bn, 1, bn)

  layers = [
      (conv1_W1, conv1_b1, conv1_W2, conv1_b2),
      (conv2_W1, conv2_b1, conv2_W2, conv2_b2),
      (conv3_W1, conv3_b1, conv3_W2, conv3_b2),
      (conv4_W1, conv4_b1, conv4_W2, conv4_b2),
  ]

  h = x
  for li, (w1, b1, w2, b2) in enumerate(layers):
    part = _sc_segment_sum(h, src_r, dst_r, zeros_init, n_chunks)
    b1r = b1.reshape(1, -1)
    b2r = b2.reshape(1, -1)
    if li < 3:
      h = _tc_layer(h, part, w1, b1r, w2, b2r, bn)
    else:
      out = _tc_layer4_pool(h, part, w1, b1r, w2, b2r, batch_r,
                            fc_W, fc_b.reshape(1, -1), bn, n_graphs)
  return out


# speedup vs baseline: 1.1555x; 1.1555x over previous
import functools

import jax
import jax.numpy as jnp
from jax import lax
from jax.experimental import pallas as pl
from jax.experimental.pallas import tpu as pltpu
from jax.experimental.pallas import tpu_sc as plsc

NC = 2
NS = 16
NW = NC * NS
K = 128


def _sc_segment_sum(h, src_r, dst_r, zeros_init, n_chunks):
  n, d = h.shape
  rows_per_sub = n // NS
  mesh = plsc.VectorSubcoreMesh(core_axis_name="c", subcore_axis_name="s")

  n_q = n_chunks // 4

  @functools.partial(
      pl.kernel,
      out_type=jax.ShapeDtypeStruct((NC, n, d), jnp.float32),
      mesh=mesh,
      scratch_types=[
          pltpu.VMEM((n_q, K), jnp.int32),
          pltpu.VMEM((n_q, K), jnp.int32),
          pltpu.VMEM((n_q, K), jnp.int32),
          pltpu.VMEM((n_q, K), jnp.int32),
          pltpu.VMEM((K, d), jnp.float32),
          pltpu.VMEM((K, d), jnp.float32),
          pltpu.VMEM_SHARED((n, d), jnp.float32),
          pltpu.SemaphoreType.DMA,
          pltpu.SemaphoreType.DMA,
          pltpu.SemaphoreType.DMA,
          pltpu.SemaphoreType.DMA,
      ],
  )
  def seg(h_hbm, src_hbm, dst_hbm, z_hbm, out_hbm, src_a, dst_a, src_b,
          dst_b, rows0, rows1, agg_sh, sia, sib, s0, s1):
    cid = lax.axis_index("c")
    sid = lax.axis_index("s")
    wid = sid * NC + cid
    pltpu.async_copy(src_hbm.at[wid, 0], src_a, sia)
    pltpu.async_copy(dst_hbm.at[wid, 0], dst_a, sia)
    pltpu.async_copy(src_hbm.at[wid, 1], src_b, sib)
    pltpu.async_copy(dst_hbm.at[wid, 1], dst_b, sib)
    pltpu.sync_copy(z_hbm, agg_sh.at[pl.ds(sid * rows_per_sub, rows_per_sub)])
    plsc.subcore_barrier()

    def process_quarter(q, src_q, dst_q, si_q):
      pltpu.make_async_copy(src_hbm.at[wid, q], src_q, si_q).wait()
      pltpu.make_async_copy(dst_hbm.at[wid, q], dst_q, si_q).wait()
      pltpu.async_copy(h_hbm.at[src_q.at[0]], rows0, s0)

      @pl.loop(0, n_q - 2, step=2)
      def _(j):
        pltpu.async_copy(h_hbm.at[src_q.at[j + 1]], rows1, s1)
        pltpu.make_async_copy(h_hbm.at[src_q.at[j]], rows0, s0).wait()
        pltpu.sync_copy(rows0, agg_sh.at[dst_q.at[j]], add=True)
        pltpu.async_copy(h_hbm.at[src_q.at[j + 2]], rows0, s0)
        pltpu.make_async_copy(h_hbm.at[src_q.at[j + 1]], rows1, s1).wait()
        pltpu.sync_copy(rows1, agg_sh.at[dst_q.at[j + 1]], add=True)

      jl = n_q - 2
      pltpu.async_copy(h_hbm.at[src_q.at[jl + 1]], rows1, s1)
      pltpu.make_async_copy(h_hbm.at[src_q.at[jl]], rows0, s0).wait()
      pltpu.sync_copy(rows0, agg_sh.at[dst_q.at[jl]], add=True)
      pltpu.make_async_copy(h_hbm.at[src_q.at[jl + 1]], rows1, s1).wait()
      pltpu.sync_copy(rows1, agg_sh.at[dst_q.at[jl + 1]], add=True)

    process_quarter(0, src_a, dst_a, sia)
    pltpu.async_copy(src_hbm.at[wid, 2], src_a, sia)
    pltpu.async_copy(dst_hbm.at[wid, 2], dst_a, sia)
    process_quarter(1, src_b, dst_b, sib)
    pltpu.async_copy(src_hbm.at[wid, 3], src_b, sib)
    pltpu.async_copy(dst_hbm.at[wid, 3], dst_b, sib)
    process_quarter(2, src_a, dst_a, sia)
    process_quarter(3, src_b, dst_b, sib)

    plsc.subcore_barrier()
    sl = pl.ds(sid * rows_per_sub, rows_per_sub)
    pltpu.sync_copy(agg_sh.at[sl], out_hbm.at[cid, sl])

  return seg(h, src_r, dst_r, zeros_init)


def _tc_layer(h, partial, w1, b1, w2, b2, bn):
  n, d = h.shape
  grid = n // bn

  def body(h_ref, p_ref, w1_ref, b1_ref, w2_ref, b2_ref, o_ref):
    z = h_ref[...] + p_ref[0] + p_ref[1]
    z = jnp.maximum(
        jnp.dot(z, w1_ref[...], preferred_element_type=jnp.float32)
        + b1_ref[...], 0.0)
    o = jnp.dot(z, w2_ref[...], preferred_element_type=jnp.float32) \
        + b2_ref[...]
    o_ref[...] = jnp.maximum(o, 0.0)

  return pl.pallas_call(
      body,
      grid=(grid,),
      in_specs=[
          pl.BlockSpec((bn, d), lambda i: (i, 0)),
          pl.BlockSpec((NC, bn, d), lambda i: (0, i, 0)),
          pl.BlockSpec((d, d), lambda i: (0, 0)),
          pl.BlockSpec((1, d), lambda i: (0, 0)),
          pl.BlockSpec((d, d), lambda i: (0, 0)),
          pl.BlockSpec((1, d), lambda i: (0, 0)),
      ],
      out_specs=pl.BlockSpec((bn, d), lambda i: (i, 0)),
      out_shape=jax.ShapeDtypeStruct((n, d), jnp.float32),
  )(h, partial, w1, b1, w2, b2)


def _tc_layer4_pool(h, partial, w1, b1, w2, b2, batch_r, fc_w, fc_b, bn,
                    n_graphs):
  n, d = h.shape
  c = fc_w.shape[1]
  grid = n // bn

  def body(h_ref, p_ref, w1_ref, b1_ref, w2_ref, b2_ref, batch_ref,
           fcw_ref, fcb_ref, o_ref, sums_scr, cnts_scr):
    i = pl.program_id(0)

    @pl.when(i == 0)
    def _():
      sums_scr[...] = jnp.zeros_like(sums_scr)
      cnts_scr[...] = jnp.zeros_like(cnts_scr)

    z = h_ref[...] + p_ref[0] + p_ref[1]
    z = jnp.maximum(
        jnp.dot(z, w1_ref[...], preferred_element_type=jnp.float32)
        + b1_ref[...], 0.0)
    h4 = jnp.maximum(
        jnp.dot(z, w2_ref[...], preferred_element_type=jnp.float32)
        + b2_ref[...], 0.0)

    b2d = batch_ref[0]
    mask_t = (lax.broadcasted_iota(jnp.int32, (n_graphs, bn), 0)
              == b2d).astype(jnp.float32)
    sums_scr[...] += jnp.dot(mask_t, h4, preferred_element_type=jnp.float32)
    cnts_scr[...] += jnp.dot(mask_t, jnp.ones((bn, d), jnp.float32),
                             preferred_element_type=jnp.float32)

    @pl.when(i == grid - 1)
    def _():
      pooled = sums_scr[...] / jnp.maximum(cnts_scr[...], 1.0)
      logits = jnp.dot(pooled, fcw_ref[...],
                       preferred_element_type=jnp.float32) + fcb_ref[...]
      m = jnp.max(logits, axis=1, keepdims=True)
      e = jnp.exp(logits - m)
      s = jnp.sum(e, axis=1, keepdims=True)
      o_ref[...] = logits - m - jnp.log(s)

  return pl.pallas_call(
      body,
      grid=(grid,),
      in_specs=[
          pl.BlockSpec((bn, d), lambda i: (i, 0)),
          pl.BlockSpec((NC, bn, d), lambda i: (0, i, 0)),
          pl.BlockSpec((d, d), lambda i: (0, 0)),
          pl.BlockSpec((1, d), lambda i: (0, 0)),
          pl.BlockSpec((d, d), lambda i: (0, 0)),
          pl.BlockSpec((1, d), lambda i: (0, 0)),
          pl.BlockSpec((1, 1, bn), lambda i: (i, 0, 0)),
          pl.BlockSpec((d, c), lambda i: (0, 0)),
          pl.BlockSpec((1, c), lambda i: (0, 0)),
      ],
      out_specs=pl.BlockSpec((n_graphs, c), lambda i: (0, 0)),
      out_shape=jax.ShapeDtypeStruct((n_graphs, c), jnp.float32),
      scratch_shapes=[
          pltpu.VMEM((n_graphs, d), jnp.float32),
          pltpu.VMEM((n_graphs, d), jnp.float32),
      ],
  )(h, partial, w1, b1, w2, b2, batch_r, fc_w, fc_b)


def kernel(x, edge_index, batch,
           conv1_W1, conv1_b1, conv1_W2, conv1_b2,
           conv2_W1, conv2_b1, conv2_W2, conv2_b2,
           conv3_W1, conv3_b1, conv3_W2, conv3_b2,
           conv4_W1, conv4_b1, conv4_W2, conv4_b2,
           fc_W, fc_b):
  n, d = x.shape
  e = edge_index.shape[1]
  n_graphs = 64
  n_pad = ((n + NS * 8 * 4 - 1) // (NS * 8 * 4)) * (NS * 8 * 4)
  if n_pad == n:
    n_pad = n
  x = jnp.pad(x, ((0, n_pad - n), (0, 0)))
  batch = jnp.pad(batch.astype(jnp.int32), (0, n_pad - n),
                  constant_values=n_graphs)
  bn = n_pad // 10
  rows_per_sub = n_pad // NS

  n_chunks = -(-e // (NW * K))
  n_chunks = ((n_chunks + 7) // 8) * 8
  e_pad = NW * n_chunks * K
  src = edge_index[0].astype(jnp.int32)
  dst = edge_index[1].astype(jnp.int32)
  pad = e_pad - e
  if pad:
    r = jnp.arange(pad, dtype=jnp.int32)
    src = jnp.concatenate([src, r % n])
    dst = jnp.concatenate([dst, n + r % (n_pad - n)])
  src_r = src.reshape(NW, 4, n_chunks // 4, K)
  dst_r = dst.reshape(NW, 4, n_chunks // 4, K)

  zeros_init = jnp.zeros((rows_per_sub, d), jnp.float32)
  batch_r = batch.reshape(n_pad // bn, 1, bn)

  layers = [
      (conv1_W1, conv1_b1, conv1_W2, conv1_b2),
      (conv2_W1, conv2_b1, conv2_W2, conv2_b2),
      (conv3_W1, conv3_b1, conv3_W2, conv3_b2),
      (conv4_W1, conv4_b1, conv4_W2, conv4_b2),
  ]

  h = x
  for li, (w1, b1, w2, b2) in enumerate(layers):
    part = _sc_segment_sum(h, src_r, dst_r, zeros_init, n_chunks)
    b1r = b1.reshape(1, -1)
    b2r = b2.reshape(1, -1)
    if li < 3:
      h = _tc_layer(h, part, w1, b1r, w2, b2r, bn)
    else:
      out = _tc_layer4_pool(h, part, w1, b1r, w2, b2r, batch_r,
                            fc_W, fc_b.reshape(1, -1), bn, n_graphs)
  return out
